# speedup vs baseline: 1.0198x; 1.0198x over previous
import jax
import jax.numpy as jnp
from jax.experimental import pallas as pl
from jax.experimental.pallas import tpu as pltpu

N = 10000
B = 64
IN_SIZE = 128
EMB = 128
ROW_TILE = 400
NUM_TILES = N // ROW_TILE


def _body(a_ref, nbt_ref, x_ref, w_in_t_ref, b_in_ref, w_gcn_ref, w_g_t_ref,
          b_g_ref, hs_ref, hg_ref, h_vmem, pooled):
    i = pl.program_id(0)

    @pl.when(i == 0)
    def _init():
        h_vmem[...] = jnp.maximum(
            jnp.dot(x_ref[...], w_in_t_ref[...],
                    preferred_element_type=jnp.float32) + b_in_ref[...],
            0.0)
        pooled[...] = jnp.zeros_like(pooled)

    z = jnp.dot(a_ref[...], h_vmem[...], preferred_element_type=jnp.float32)
    z = jnp.dot(z, w_gcn_ref[...], preferred_element_type=jnp.float32)
    r = jnp.maximum(z, 0.0)
    nrm = jnp.sqrt(jnp.sum(r * r, axis=-1, keepdims=True))
    hs = r / jnp.maximum(nrm, 1e-12)
    hs_ref[...] = hs

    pooled[...] += jax.lax.dot_general(
        nbt_ref[...], hs, (((0,), (0,)), ((), ())),
        preferred_element_type=jnp.float32)

    @pl.when(i == NUM_TILES - 1)
    def _finish():
        g = jnp.dot(pooled[...], w_g_t_ref[...],
                    preferred_element_type=jnp.float32) + b_g_ref[...]
        g = jnp.maximum(g, 0.0)
        nrm2 = jnp.sqrt(jnp.sum(g * g, axis=-1, keepdims=True))
        hg_ref[...] = g / jnp.maximum(nrm2, 1e-12)


@jax.jit
def kernel(node_matrix, node_batch, input_node_features, W_in, b_in, W_gcn,
           W_g, b_g):
    nbt = node_batch.T
    b_in2 = b_in.reshape(1, EMB)
    b_g2 = b_g.reshape(1, EMB)
    w_in_t = W_in.T
    w_g_t = W_g.T

    grid = (NUM_TILES,)
    h_struct, h_graph = pl.pallas_call(
        _body,
        grid=grid,
        in_specs=[
            pl.BlockSpec((ROW_TILE, N), lambda i: (i, 0)),
            pl.BlockSpec((ROW_TILE, B), lambda i: (i, 0)),
            pl.BlockSpec((N, IN_SIZE), lambda i: (0, 0)),
            pl.BlockSpec((IN_SIZE, EMB), lambda i: (0, 0)),
            pl.BlockSpec((1, EMB), lambda i: (0, 0)),
            pl.BlockSpec((EMB, EMB), lambda i: (0, 0)),
            pl.BlockSpec((EMB, EMB), lambda i: (0, 0)),
            pl.BlockSpec((1, EMB), lambda i: (0, 0)),
        ],
        out_specs=[
            pl.BlockSpec((ROW_TILE, EMB), lambda i: (i, 0)),
            pl.BlockSpec((B, EMB), lambda i: (0, 0)),
        ],
        out_shape=[
            jax.ShapeDtypeStruct((N, EMB), jnp.float32),
            jax.ShapeDtypeStruct((B, EMB), jnp.float32),
        ],
        scratch_shapes=[
            pltpu.VMEM((N, EMB), jnp.float32),
            pltpu.VMEM((B, EMB), jnp.float32),
        ],
    )(node_matrix, nbt, input_node_features, w_in_t, b_in2, W_gcn, w_g_t,
      b_g2)
    return (h_struct, h_graph)
